# initial kernel scaffold (unmeasured)
import jax
import jax.numpy as jnp
from jax import lax
from jax.experimental import pallas as pl
from jax.experimental.pallas import tpu as pltpu

N_DEV = 32
FP8 = jnp.float8_e4m3fn


def _allgather_body(x_ref, w_ref, xg_ref, wg_ref, sem_sx, sem_rx, sem_sw, sem_rw):
    my = lax.axis_index("i")
    left = lax.rem(my + (N_DEV - 1), N_DEV)
    right = lax.rem(my + 1, N_DEV)
    k_per = x_ref.shape[1]

    barrier = pltpu.get_barrier_semaphore()
    pl.semaphore_signal(barrier, inc=1, device_id=(left,),
                        device_id_type=pl.DeviceIdType.MESH)
    pl.semaphore_signal(barrier, inc=1, device_id=(right,),
                        device_id_type=pl.DeviceIdType.MESH)
    pl.semaphore_wait(barrier, 2)

    my_off = my * k_per
    xg_ref[:, pl.ds(my_off, k_per)] = x_ref[...].astype(FP8)
    wg_ref[pl.ds(my_off, k_per), :] = w_ref[...].astype(FP8)

    for h in range(N_DEV - 1):
        src = lax.rem(my + (N_DEV - h), N_DEV)
        off = src * k_per
        rx = pltpu.make_async_remote_copy(
            src_ref=xg_ref.at[:, pl.ds(off, k_per)],
            dst_ref=xg_ref.at[:, pl.ds(off, k_per)],
            send_sem=sem_sx, recv_sem=sem_rx,
            device_id=(right,), device_id_type=pl.DeviceIdType.MESH,
        )
        rw = pltpu.make_async_remote_copy(
            src_ref=wg_ref.at[pl.ds(off, k_per), :],
            dst_ref=wg_ref.at[pl.ds(off, k_per), :],
            send_sem=sem_sw, recv_sem=sem_rw,
            device_id=(right,), device_id_type=pl.DeviceIdType.MESH,
        )
        rx.start()
        rw.start()
        rx.wait()
        rw.wait()


def _gemm_body(s_ref, xg_ref, wg_ref, o_ref):
    acc = jnp.dot(xg_ref[...], wg_ref[...], preferred_element_type=jnp.float32)
    o_ref[...] = jnp.maximum(acc * s_ref[0, 0], 0.0)


def kernel(x, w_mat, scale_x, scale_w):
    m, k_per = x.shape
    _, n = w_mat.shape
    k = k_per * N_DEV

    xg, wg = pl.pallas_call(
        _allgather_body,
        out_shape=[
            jax.ShapeDtypeStruct((m, k), FP8),
            jax.ShapeDtypeStruct((k, n), FP8),
        ],
        in_specs=[
            pl.BlockSpec(memory_space=pltpu.VMEM),
            pl.BlockSpec(memory_space=pltpu.VMEM),
        ],
        out_specs=[
            pl.BlockSpec(memory_space=pltpu.VMEM),
            pl.BlockSpec(memory_space=pltpu.VMEM),
        ],
        scratch_shapes=[
            pltpu.SemaphoreType.DMA,
            pltpu.SemaphoreType.DMA,
            pltpu.SemaphoreType.DMA,
            pltpu.SemaphoreType.DMA,
        ],
        compiler_params=pltpu.CompilerParams(collective_id=0),
    )(x, w_mat)

    s = (scale_x.astype(jnp.float32) * scale_w.astype(jnp.float32)).reshape(1, 1)

    m_blk = 256
    out = pl.pallas_call(
        _gemm_body,
        grid=(m // m_blk,),
        out_shape=jax.ShapeDtypeStruct((m, n), jnp.float32),
        in_specs=[
            pl.BlockSpec((1, 1), lambda i: (0, 0), memory_space=pltpu.SMEM),
            pl.BlockSpec((m_blk, k), lambda i: (i, 0)),
            pl.BlockSpec((k, n), lambda i: (0, 0)),
        ],
        out_specs=pl.BlockSpec((m_blk, n), lambda i: (i, 0)),
    )(s, xg, wg)
    return out


# baseline (device time: 760014 ns/iter reference)
import jax
import jax.numpy as jnp
from jax import lax
from jax.experimental import pallas as pl
from jax.experimental.pallas import tpu as pltpu

N_DEV = 32
FP8 = jnp.float8_e4m3fn


def _allgather_body(x_ref, w_ref, xg_ref, wg_ref, xs_ref, ws_ref,
                    sem_cx, sem_cw, sem_sx, sem_rx, sem_sw, sem_rw):
    my = lax.axis_index("i")
    left = lax.rem(my + (N_DEV - 1), N_DEV)
    right = lax.rem(my + 1, N_DEV)
    k_per = x_ref.shape[1]

    my_off = my * k_per
    xs_ref[...] = x_ref[...].astype(FP8)
    ws_ref[...] = w_ref[...].astype(FP8)
    cx = pltpu.make_async_copy(xs_ref, xg_ref.at[:, pl.ds(my_off, k_per)], sem_cx)
    cw = pltpu.make_async_copy(ws_ref, wg_ref.at[pl.ds(my_off, k_per), :], sem_cw)
    cx.start()
    cw.start()
    cx.wait()
    cw.wait()

    barrier = pltpu.get_barrier_semaphore()
    pl.semaphore_signal(barrier, inc=1, device_id=(left,),
                        device_id_type=pl.DeviceIdType.MESH)
    pl.semaphore_signal(barrier, inc=1, device_id=(right,),
                        device_id_type=pl.DeviceIdType.MESH)
    pl.semaphore_wait(barrier, 2)

    for h in range(N_DEV - 1):
        src = lax.rem(my + (N_DEV - h), N_DEV)
        off = src * k_per
        rx = pltpu.make_async_remote_copy(
            src_ref=xg_ref.at[:, pl.ds(off, k_per)],
            dst_ref=xg_ref.at[:, pl.ds(off, k_per)],
            send_sem=sem_sx, recv_sem=sem_rx,
            device_id=(right,), device_id_type=pl.DeviceIdType.MESH,
        )
        rw = pltpu.make_async_remote_copy(
            src_ref=wg_ref.at[pl.ds(off, k_per), :],
            dst_ref=wg_ref.at[pl.ds(off, k_per), :],
            send_sem=sem_sw, recv_sem=sem_rw,
            device_id=(right,), device_id_type=pl.DeviceIdType.MESH,
        )
        rx.start()
        rw.start()
        rx.wait()
        rw.wait()


def _gemm_body(s_ref, xg_ref, wg_ref, o_ref):
    acc = jnp.dot(xg_ref[...], wg_ref[...], preferred_element_type=jnp.float32)
    o_ref[...] = jnp.maximum(acc * s_ref[0, 0], 0.0)


def kernel(x, w_mat, scale_x, scale_w):
    m, k_per = x.shape
    _, n = w_mat.shape
    k = k_per * N_DEV

    xg, wg = pl.pallas_call(
        _allgather_body,
        out_shape=[
            jax.ShapeDtypeStruct((m, k), FP8),
            jax.ShapeDtypeStruct((k, n), FP8),
        ],
        in_specs=[
            pl.BlockSpec(memory_space=pltpu.VMEM),
            pl.BlockSpec(memory_space=pltpu.VMEM),
        ],
        out_specs=[
            pl.BlockSpec(memory_space=pl.ANY),
            pl.BlockSpec(memory_space=pl.ANY),
        ],
        scratch_shapes=[
            pltpu.VMEM((m, k_per), FP8),
            pltpu.VMEM((k_per, n), FP8),
            pltpu.SemaphoreType.DMA,
            pltpu.SemaphoreType.DMA,
            pltpu.SemaphoreType.DMA,
            pltpu.SemaphoreType.DMA,
            pltpu.SemaphoreType.DMA,
            pltpu.SemaphoreType.DMA,
        ],
        compiler_params=pltpu.CompilerParams(collective_id=0),
    )(x, w_mat)

    s = (scale_x.astype(jnp.float32) * scale_w.astype(jnp.float32)).reshape(1, 1)

    m_blk = 256
    out = pl.pallas_call(
        _gemm_body,
        grid=(m // m_blk,),
        out_shape=jax.ShapeDtypeStruct((m, n), jnp.float32),
        in_specs=[
            pl.BlockSpec((1, 1), lambda i: (0, 0), memory_space=pltpu.SMEM),
            pl.BlockSpec((m_blk, k), lambda i: (i, 0)),
            pl.BlockSpec((k, n), lambda i: (0, 0)),
        ],
        out_specs=pl.BlockSpec((m_blk, n), lambda i: (i, 0)),
        compiler_params=pltpu.CompilerParams(
            vmem_limit_bytes=60 * 1024 * 1024,
        ),
    )(s, xg, wg)
    return out


# device time: 757272 ns/iter; 1.0036x vs baseline; 1.0036x over previous
import jax
import jax.numpy as jnp
from jax import lax
from jax.experimental import pallas as pl
from jax.experimental.pallas import tpu as pltpu

N_DEV = 32
FP8 = jnp.float8_e4m3fn


def _allgather_body(x_ref, w_ref, xg_ref, wg_ref, xs_ref, ws_ref,
                    sem_cx, sem_cw, sems_s, sems_r):
    my = lax.axis_index("i")
    left = lax.rem(my + (N_DEV - 1), N_DEV)
    right = lax.rem(my + 1, N_DEV)
    m, k_per = x_ref.shape
    n = w_ref.shape[1]
    m2 = m // 2
    n2 = n // 2

    my_off = my * k_per
    xs_ref[...] = x_ref[...].astype(FP8)
    ws_ref[...] = w_ref[...].astype(FP8)
    cx = pltpu.make_async_copy(xs_ref, xg_ref.at[:, pl.ds(my_off, k_per)], sem_cx)
    cw = pltpu.make_async_copy(ws_ref, wg_ref.at[pl.ds(my_off, k_per), :], sem_cw)
    cx.start()
    cw.start()
    cx.wait()
    cw.wait()

    barrier = pltpu.get_barrier_semaphore()
    pl.semaphore_signal(barrier, inc=1, device_id=(left,),
                        device_id_type=pl.DeviceIdType.MESH)
    pl.semaphore_signal(barrier, inc=1, device_id=(right,),
                        device_id_type=pl.DeviceIdType.MESH)
    pl.semaphore_wait(barrier, 2)

    for h in range(N_DEV - 1):
        cw_off = lax.rem(my + (N_DEV - h), N_DEV) * k_per
        ccw_off = lax.rem(my + h, N_DEV) * k_per
        rdmas = []
        for (sl, dev, si) in (
            ((pl.ds(0, m2), pl.ds(cw_off, k_per)), right, 0),
            ((pl.ds(m2, m2), pl.ds(ccw_off, k_per)), left, 1),
            ((pl.ds(cw_off, k_per), pl.ds(0, n2)), right, 2),
            ((pl.ds(ccw_off, k_per), pl.ds(n2, n2)), left, 3),
        ):
            ref = xg_ref if si < 2 else wg_ref
            r = pltpu.make_async_remote_copy(
                src_ref=ref.at[sl[0], sl[1]],
                dst_ref=ref.at[sl[0], sl[1]],
                send_sem=sems_s.at[si], recv_sem=sems_r.at[si],
                device_id=(dev,), device_id_type=pl.DeviceIdType.MESH,
            )
            r.start()
            rdmas.append(r)
        for r in rdmas:
            r.wait_recv()
        for r in rdmas:
            r.wait_send()


def _gemm_body(s_ref, xg_ref, wg_ref, o_ref):
    acc = jnp.dot(xg_ref[...], wg_ref[...], preferred_element_type=jnp.float32)
    o_ref[...] = jnp.maximum(acc * s_ref[0, 0], 0.0)


def kernel(x, w_mat, scale_x, scale_w):
    m, k_per = x.shape
    _, n = w_mat.shape
    k = k_per * N_DEV

    xg, wg = pl.pallas_call(
        _allgather_body,
        out_shape=[
            jax.ShapeDtypeStruct((m, k), FP8),
            jax.ShapeDtypeStruct((k, n), FP8),
        ],
        in_specs=[
            pl.BlockSpec(memory_space=pltpu.VMEM),
            pl.BlockSpec(memory_space=pltpu.VMEM),
        ],
        out_specs=[
            pl.BlockSpec(memory_space=pl.ANY),
            pl.BlockSpec(memory_space=pl.ANY),
        ],
        scratch_shapes=[
            pltpu.VMEM((m, k_per), FP8),
            pltpu.VMEM((k_per, n), FP8),
            pltpu.SemaphoreType.DMA,
            pltpu.SemaphoreType.DMA,
            pltpu.SemaphoreType.DMA((4,)),
            pltpu.SemaphoreType.DMA((4,)),
        ],
        compiler_params=pltpu.CompilerParams(collective_id=0),
    )(x, w_mat)

    s = (scale_x.astype(jnp.float32) * scale_w.astype(jnp.float32)).reshape(1, 1)

    m_blk = 256
    out = pl.pallas_call(
        _gemm_body,
        grid=(m // m_blk,),
        out_shape=jax.ShapeDtypeStruct((m, n), jnp.float32),
        in_specs=[
            pl.BlockSpec((1, 1), lambda i: (0, 0), memory_space=pltpu.SMEM),
            pl.BlockSpec((m_blk, k), lambda i: (i, 0)),
            pl.BlockSpec((k, n), lambda i: (0, 0)),
        ],
        out_specs=pl.BlockSpec((m_blk, n), lambda i: (i, 0)),
        compiler_params=pltpu.CompilerParams(
            vmem_limit_bytes=60 * 1024 * 1024,
        ),
    )(s, xg, wg)
    return out


# device time: 503721 ns/iter; 1.5088x vs baseline; 1.5034x over previous
import jax
import jax.numpy as jnp
from jax import lax
from jax.experimental import pallas as pl
from jax.experimental.pallas import tpu as pltpu

N_DEV = 32
FP8 = jnp.float8_e4m3fn

RING = [0, 3, 4, 7, 15, 12, 11, 8, 16, 19, 20, 23, 31, 28, 27, 24,
        25, 26, 29, 30, 22, 21, 18, 17, 9, 10, 13, 14, 6, 5, 2, 1]
INV_RING = [0] * N_DEV
for _r, _d in enumerate(RING):
    INV_RING[_d] = _r


def _allgather_body(nbr_ref, cw_ids_ref, ccw_ids_ref, x_ref, w_ref,
                    xg_ref, wg_ref, xs_ref, ws_ref,
                    sem_cx, sem_cw, sems_s, sems_r):
    my = lax.axis_index("i")
    right = nbr_ref[0]
    left = nbr_ref[1]
    m, k_per = x_ref.shape
    n = w_ref.shape[1]
    m2 = m // 2
    n2 = n // 2

    my_off = my * k_per
    xs_ref[...] = x_ref[...].astype(FP8)
    ws_ref[...] = w_ref[...].astype(FP8)
    cx = pltpu.make_async_copy(xs_ref, xg_ref.at[:, pl.ds(my_off, k_per)], sem_cx)
    cw = pltpu.make_async_copy(ws_ref, wg_ref.at[pl.ds(my_off, k_per), :], sem_cw)
    cx.start()
    cw.start()
    cx.wait()
    cw.wait()

    barrier = pltpu.get_barrier_semaphore()
    pl.semaphore_signal(barrier, inc=1, device_id=(left,),
                        device_id_type=pl.DeviceIdType.MESH)
    pl.semaphore_signal(barrier, inc=1, device_id=(right,),
                        device_id_type=pl.DeviceIdType.MESH)
    pl.semaphore_wait(barrier, 2)

    for h in range(N_DEV - 1):
        cw_off = cw_ids_ref[h] * k_per
        ccw_off = ccw_ids_ref[h] * k_per
        rdmas = []
        for (sl, dev, si) in (
            ((pl.ds(0, m2), pl.ds(cw_off, k_per)), right, 0),
            ((pl.ds(m2, m2), pl.ds(ccw_off, k_per)), left, 1),
            ((pl.ds(cw_off, k_per), pl.ds(0, n2)), right, 2),
            ((pl.ds(ccw_off, k_per), pl.ds(n2, n2)), left, 3),
        ):
            ref = xg_ref if si < 2 else wg_ref
            r = pltpu.make_async_remote_copy(
                src_ref=ref.at[sl[0], sl[1]],
                dst_ref=ref.at[sl[0], sl[1]],
                send_sem=sems_s.at[si], recv_sem=sems_r.at[si],
                device_id=(dev,), device_id_type=pl.DeviceIdType.MESH,
            )
            r.start()
            rdmas.append(r)
        for r in rdmas:
            r.wait_recv()
        for r in rdmas:
            r.wait_send()


def _gemm_body(s_ref, xg_ref, wg_ref, o_ref):
    acc = jnp.dot(xg_ref[...], wg_ref[...], preferred_element_type=jnp.float32)
    o_ref[...] = jnp.maximum(acc * s_ref[0, 0], 0.0)


def kernel(x, w_mat, scale_x, scale_w):
    m, k_per = x.shape
    _, n = w_mat.shape
    k = k_per * N_DEV

    ring = jnp.asarray(RING, dtype=jnp.int32)
    inv = jnp.asarray(INV_RING, dtype=jnp.int32)
    r = inv[lax.axis_index("i")]
    idx = jnp.arange(N_DEV, dtype=jnp.int32)
    cw_ids = ring[jnp.mod(r - idx, N_DEV)]
    ccw_ids = ring[jnp.mod(r + idx, N_DEV)]
    nbr = jnp.stack([ring[jnp.mod(r + 1, N_DEV)], ring[jnp.mod(r - 1, N_DEV)]])

    xg, wg = pl.pallas_call(
        _allgather_body,
        out_shape=[
            jax.ShapeDtypeStruct((m, k), FP8),
            jax.ShapeDtypeStruct((k, n), FP8),
        ],
        in_specs=[
            pl.BlockSpec(memory_space=pltpu.SMEM),
            pl.BlockSpec(memory_space=pltpu.SMEM),
            pl.BlockSpec(memory_space=pltpu.SMEM),
            pl.BlockSpec(memory_space=pltpu.VMEM),
            pl.BlockSpec(memory_space=pltpu.VMEM),
        ],
        out_specs=[
            pl.BlockSpec(memory_space=pl.ANY),
            pl.BlockSpec(memory_space=pl.ANY),
        ],
        scratch_shapes=[
            pltpu.VMEM((m, k_per), FP8),
            pltpu.VMEM((k_per, n), FP8),
            pltpu.SemaphoreType.DMA,
            pltpu.SemaphoreType.DMA,
            pltpu.SemaphoreType.DMA((4,)),
            pltpu.SemaphoreType.DMA((4,)),
        ],
        compiler_params=pltpu.CompilerParams(collective_id=0),
    )(nbr, cw_ids, ccw_ids, x, w_mat)

    s = (scale_x.astype(jnp.float32) * scale_w.astype(jnp.float32)).reshape(1, 1)

    m_blk = 256
    out = pl.pallas_call(
        _gemm_body,
        grid=(m // m_blk,),
        out_shape=jax.ShapeDtypeStruct((m, n), jnp.float32),
        in_specs=[
            pl.BlockSpec((1, 1), lambda i: (0, 0), memory_space=pltpu.SMEM),
            pl.BlockSpec((m_blk, k), lambda i: (i, 0)),
            pl.BlockSpec((k, n), lambda i: (0, 0)),
        ],
        out_specs=pl.BlockSpec((m_blk, n), lambda i: (i, 0)),
        compiler_params=pltpu.CompilerParams(
            vmem_limit_bytes=60 * 1024 * 1024,
        ),
    )(s, xg, wg)
    return out


# device time: 439674 ns/iter; 1.7286x vs baseline; 1.1457x over previous
import jax
import jax.numpy as jnp
from jax import lax
from jax.experimental import pallas as pl
from jax.experimental.pallas import tpu as pltpu

N_DEV = 32
FP8 = jnp.float8_e4m3fn

RING = [0, 3, 4, 7, 15, 12, 11, 8, 16, 19, 20, 23, 31, 28, 27, 24,
        25, 26, 29, 30, 22, 21, 18, 17, 9, 10, 13, 14, 6, 5, 2, 1]
INV_RING = [0] * N_DEV
for _r, _d in enumerate(RING):
    INV_RING[_d] = _r


def _allgather_body(nbr_ref, cw_ids_ref, ccw_ids_ref, x_ref, w_ref,
                    xg_ref, wg_ref, xs_ref, ws_ref,
                    sem_cx, sem_cw, sems_s, sems_r):
    my = lax.axis_index("i")
    right = nbr_ref[0]
    left = nbr_ref[1]
    m, k_per = x_ref.shape
    n = w_ref.shape[1]
    m2 = m // 2
    n2 = n // 2

    my_off = my * k_per
    xs_ref[...] = x_ref[...].astype(FP8)
    ws_ref[...] = w_ref[...].astype(FP8)
    cx = pltpu.make_async_copy(xs_ref, xg_ref.at[:, pl.ds(my_off, k_per)], sem_cx)
    cw = pltpu.make_async_copy(ws_ref, wg_ref.at[pl.ds(my_off, k_per), :], sem_cw)
    cx.start()
    cw.start()
    cx.wait()
    cw.wait()

    barrier = pltpu.get_barrier_semaphore()
    pl.semaphore_signal(barrier, inc=1, device_id=(left,),
                        device_id_type=pl.DeviceIdType.MESH)
    pl.semaphore_signal(barrier, inc=1, device_id=(right,),
                        device_id_type=pl.DeviceIdType.MESH)
    pl.semaphore_wait(barrier, 2)

    def _mk(si, h, chunk_id, send):
        off = chunk_id * k_per
        if si == 0:
            ref, sl, dev = wg_ref, (pl.ds(off, k_per), pl.ds(0, n2)), right
        elif si == 1:
            ref, sl, dev = wg_ref, (pl.ds(off, k_per), pl.ds(n2, n2)), left
        elif si == 2:
            ref, sl, dev = xg_ref, (pl.ds(0, m2), pl.ds(off, k_per)), right
        else:
            ref, sl, dev = xg_ref, (pl.ds(m2, m2), pl.ds(off, k_per)), left
        del send
        return pltpu.make_async_remote_copy(
            src_ref=ref.at[sl[0], sl[1]],
            dst_ref=ref.at[sl[0], sl[1]],
            send_sem=sems_s.at[h % 2, si], recv_sem=sems_r.at[h % 2, si],
            device_id=(dev,), device_id_type=pl.DeviceIdType.MESH,
        )

    prev_sends = None
    for h in range(N_DEV - 1):
        cw_snd, ccw_snd = cw_ids_ref[h], ccw_ids_ref[h]
        sends = []
        for si in range(4):
            if h > 0:
                _mk(si, h - 1, cw_snd if si % 2 == 0 else ccw_snd,
                    send=False).wait_recv()
            s = _mk(si, h, cw_snd if si % 2 == 0 else ccw_snd, send=True)
            s.start()
            sends.append(s)
        if prev_sends is not None:
            for s in prev_sends:
                s.wait_send()
        prev_sends = sends

    for si in range(4):
        _mk(si, N_DEV - 2,
            cw_ids_ref[N_DEV - 1] if si % 2 == 0 else ccw_ids_ref[N_DEV - 1],
            send=False).wait_recv()
    for s in prev_sends:
        s.wait_send()


def _gemm_body(s_ref, xg_ref, wg_ref, o_ref):
    acc = jnp.dot(xg_ref[...], wg_ref[...], preferred_element_type=jnp.float32)
    o_ref[...] = jnp.maximum(acc * s_ref[0, 0], 0.0)


def kernel(x, w_mat, scale_x, scale_w):
    m, k_per = x.shape
    _, n = w_mat.shape
    k = k_per * N_DEV

    ring = jnp.asarray(RING, dtype=jnp.int32)
    inv = jnp.asarray(INV_RING, dtype=jnp.int32)
    r = inv[lax.axis_index("i")]
    idx = jnp.arange(N_DEV, dtype=jnp.int32)
    cw_ids = ring[jnp.mod(r - idx, N_DEV)]
    ccw_ids = ring[jnp.mod(r + idx, N_DEV)]
    nbr = jnp.stack([ring[jnp.mod(r + 1, N_DEV)], ring[jnp.mod(r - 1, N_DEV)]])

    xg, wg = pl.pallas_call(
        _allgather_body,
        out_shape=[
            jax.ShapeDtypeStruct((m, k), FP8),
            jax.ShapeDtypeStruct((k, n), FP8),
        ],
        in_specs=[
            pl.BlockSpec(memory_space=pltpu.SMEM),
            pl.BlockSpec(memory_space=pltpu.SMEM),
            pl.BlockSpec(memory_space=pltpu.SMEM),
            pl.BlockSpec(memory_space=pltpu.VMEM),
            pl.BlockSpec(memory_space=pltpu.VMEM),
        ],
        out_specs=[
            pl.BlockSpec(memory_space=pl.ANY),
            pl.BlockSpec(memory_space=pl.ANY),
        ],
        scratch_shapes=[
            pltpu.VMEM((m, k_per), FP8),
            pltpu.VMEM((k_per, n), FP8),
            pltpu.SemaphoreType.DMA,
            pltpu.SemaphoreType.DMA,
            pltpu.SemaphoreType.DMA((2, 4)),
            pltpu.SemaphoreType.DMA((2, 4)),
        ],
        compiler_params=pltpu.CompilerParams(collective_id=0),
    )(nbr, cw_ids, ccw_ids, x, w_mat)

    s = (scale_x.astype(jnp.float32) * scale_w.astype(jnp.float32)).reshape(1, 1)

    m_blk = 256
    out = pl.pallas_call(
        _gemm_body,
        grid=(m // m_blk,),
        out_shape=jax.ShapeDtypeStruct((m, n), jnp.float32),
        in_specs=[
            pl.BlockSpec((1, 1), lambda i: (0, 0), memory_space=pltpu.SMEM),
            pl.BlockSpec((m_blk, k), lambda i: (i, 0)),
            pl.BlockSpec((k, n), lambda i: (0, 0)),
        ],
        out_specs=pl.BlockSpec((m_blk, n), lambda i: (i, 0)),
        compiler_params=pltpu.CompilerParams(
            vmem_limit_bytes=60 * 1024 * 1024,
        ),
    )(s, xg, wg)
    return out
